# baseline (speedup 1.0000x reference)
import functools

import jax
import jax.numpy as jnp
from jax.experimental import pallas as pl
from jax.experimental.pallas import tpu as pltpu

SEQ_LEN = 4096
BSZ = 2
EMBED_DIM = 4096
N_EXPERTS = 31
TOP_K = 3
ALPHA = 0.001

E_PAD = 128
ROW_BLOCK = 1024
N_ROWS = SEQ_LEN * BSZ
N_BLOCKS = N_ROWS // ROW_BLOCK
NEG = -1e30


def _gate_kernel(x_ref, w_ref, idx_ref, wgt_ref, aux_ref, cnt_acc, sum_acc):
    i = pl.program_id(0)

    @pl.when(i == 0)
    def _init():
        cnt_acc[...] = jnp.zeros_like(cnt_acc)
        sum_acc[...] = jnp.zeros_like(sum_acc)

    logits = jax.lax.dot_general(
        x_ref[...], w_ref[...],
        dimension_numbers=(((1,), (0,)), ((), ())),
        preferred_element_type=jnp.float32,
        precision=jax.lax.Precision.DEFAULT,
    )
    lane = jax.lax.broadcasted_iota(jnp.int32, (ROW_BLOCK, E_PAD), 1)
    logits = jnp.where(lane < N_EXPERTS, logits, NEG)

    m = jnp.max(logits, axis=-1, keepdims=True)
    p = jnp.exp(logits - m)
    scores = p / jnp.sum(p, axis=-1, keepdims=True)

    cur = scores
    vals = []
    idxs = []
    for _ in range(TOP_K):
        v = jnp.max(cur, axis=-1, keepdims=True)
        hit = cur >= v
        ix = jnp.min(jnp.where(hit, lane, E_PAD), axis=-1, keepdims=True)
        vals.append(v)
        idxs.append(ix)
        cur = jnp.where(lane == ix, -1.0, cur)

    topv = jnp.concatenate(vals, axis=-1)
    topi = jnp.concatenate(idxs, axis=-1)
    denom = jnp.sum(topv, axis=-1, keepdims=True) + 1e-20
    idx_ref[...] = topi
    wgt_ref[...] = topv / denom

    h = (i * ROW_BLOCK) // SEQ_LEN
    onehot_h = (jax.lax.broadcasted_iota(jnp.int32, (2, 1), 0) == h).astype(jnp.float32)

    sum_acc[...] += onehot_h * jnp.sum(scores, axis=0)[None, :]
    cnt = jnp.zeros((E_PAD,), dtype=jnp.float32)
    for j in range(TOP_K):
        cnt += jnp.sum((lane == topi[:, j:j + 1]).astype(jnp.float32), axis=0)
    cnt_acc[...] += onehot_h * cnt[None, :]

    @pl.when(i == N_BLOCKS - 1)
    def _finalize():
        scale = ALPHA * (1.0 / BSZ) * N_EXPERTS / (SEQ_LEN * SEQ_LEN * TOP_K)
        aux_ref[...] = (jnp.sum(cnt_acc[...] * sum_acc[...]) * scale).reshape(1, 1)


@functools.partial(jax.jit, static_argnums=())
def _gate(x_flat, w_pad):
    idx, wgt, aux = pl.pallas_call(
        _gate_kernel,
        grid=(N_BLOCKS,),
        in_specs=[
            pl.BlockSpec((ROW_BLOCK, EMBED_DIM), lambda i: (i, 0)),
            pl.BlockSpec((EMBED_DIM, E_PAD), lambda i: (0, 0)),
        ],
        out_specs=[
            pl.BlockSpec((ROW_BLOCK, TOP_K), lambda i: (i, 0)),
            pl.BlockSpec((ROW_BLOCK, TOP_K), lambda i: (i, 0)),
            pl.BlockSpec((1, 1), lambda i: (0, 0)),
        ],
        out_shape=[
            jax.ShapeDtypeStruct((N_ROWS, TOP_K), jnp.int32),
            jax.ShapeDtypeStruct((N_ROWS, TOP_K), jnp.float32),
            jax.ShapeDtypeStruct((1, 1), jnp.float32),
        ],
        scratch_shapes=[
            pltpu.VMEM((2, E_PAD), jnp.float32),
            pltpu.VMEM((2, E_PAD), jnp.float32),
        ],
    )(x_flat, w_pad)
    return idx, wgt, aux[0, 0]


def kernel(x, weight):
    x_flat = x.reshape(-1, EMBED_DIM)
    w_pad = jnp.zeros((EMBED_DIM, E_PAD), dtype=weight.dtype).at[:, :N_EXPERTS].set(weight.T)
    return _gate(x_flat, w_pad)


# speedup vs baseline: 1.6396x; 1.6396x over previous
import functools

import jax
import jax.numpy as jnp
from jax.experimental import pallas as pl
from jax.experimental.pallas import tpu as pltpu

SEQ_LEN = 4096
BSZ = 2
EMBED_DIM = 4096
N_EXPERTS = 31
TOP_K = 3
ALPHA = 0.001

E_PAD = 128
S_BLOCK = 256
RB = S_BLOCK * BSZ
N_BLOCKS = SEQ_LEN // S_BLOCK
HALF_BLOCKS = (SEQ_LEN // 2) // S_BLOCK
NEG = -1e30


def _start_copies(x_hbm, x_buf, sem, step, slot):
    for b in range(BSZ):
        pltpu.make_async_copy(
            x_hbm.at[pl.ds(step * S_BLOCK, S_BLOCK), b, :],
            x_buf.at[slot, pl.ds(b * S_BLOCK, S_BLOCK), :],
            sem.at[slot, b],
        ).start()


def _wait_copies(x_hbm, x_buf, sem, step, slot):
    for b in range(BSZ):
        pltpu.make_async_copy(
            x_hbm.at[pl.ds(step * S_BLOCK, S_BLOCK), b, :],
            x_buf.at[slot, pl.ds(b * S_BLOCK, S_BLOCK), :],
            sem.at[slot, b],
        ).wait()


def _gate_kernel(x_hbm, w_ref, idx_ref, wgt_ref, aux_ref, x_buf, sem,
                 cnt_acc, sum_acc):
    i = pl.program_id(0)
    slot = jax.lax.rem(i, 2)

    @pl.when(i == 0)
    def _init():
        cnt_acc[...] = jnp.zeros_like(cnt_acc)
        sum_acc[...] = jnp.zeros_like(sum_acc)
        _start_copies(x_hbm, x_buf, sem, 0, 0)

    @pl.when(i + 1 < N_BLOCKS)
    def _prefetch():
        _start_copies(x_hbm, x_buf, sem, i + 1, 1 - slot)

    _wait_copies(x_hbm, x_buf, sem, i, slot)

    xb = x_buf[slot]
    logits = jax.lax.dot_general(
        xb, w_ref[...],
        dimension_numbers=(((1,), (0,)), ((), ())),
        preferred_element_type=jnp.float32,
        precision=jax.lax.Precision.DEFAULT,
    )
    lane = jax.lax.broadcasted_iota(jnp.int32, (RB, E_PAD), 1)
    logits = jnp.where(lane < N_EXPERTS, logits, NEG)

    m = jnp.max(logits, axis=-1, keepdims=True)
    p = jnp.exp(logits - m)
    scores = p / jnp.sum(p, axis=-1, keepdims=True)

    cur = scores
    vals = []
    idxs = []
    for _ in range(TOP_K):
        v = jnp.max(cur, axis=-1, keepdims=True)
        hit = cur >= v
        ix = jnp.min(jnp.where(hit, lane, E_PAD), axis=-1, keepdims=True)
        vals.append(v)
        idxs.append(ix)
        cur = jnp.where(lane == ix, -1.0, cur)

    topv = jnp.concatenate(vals, axis=-1)
    topi = jnp.concatenate(idxs, axis=-1)
    denom = jnp.sum(topv, axis=-1, keepdims=True) + 1e-20
    idx_ref[...] = topi.reshape(BSZ, S_BLOCK, TOP_K)
    wgt_ref[...] = (topv / denom).reshape(BSZ, S_BLOCK, TOP_K)

    h = i // HALF_BLOCKS
    onehot_h = (jax.lax.broadcasted_iota(jnp.int32, (2, 1), 0) == h).astype(jnp.float32)

    sum_acc[...] += onehot_h * jnp.sum(scores, axis=0)[None, :]
    cnt = jnp.zeros((E_PAD,), dtype=jnp.float32)
    for k in range(TOP_K):
        cnt += jnp.sum((lane == topi[:, k:k + 1]).astype(jnp.float32), axis=0)
    cnt_acc[...] += onehot_h * cnt[None, :]

    @pl.when(i == N_BLOCKS - 1)
    def _finalize():
        scale = ALPHA * (1.0 / BSZ) * N_EXPERTS / (SEQ_LEN * SEQ_LEN * TOP_K)
        aux_ref[...] = (jnp.sum(cnt_acc[...] * sum_acc[...]) * scale).reshape(1, 1)


@functools.partial(jax.jit, static_argnums=())
def _gate(x, w_pad):
    idx3, wgt3, aux = pl.pallas_call(
        _gate_kernel,
        grid=(N_BLOCKS,),
        in_specs=[
            pl.BlockSpec(memory_space=pl.ANY),
            pl.BlockSpec((EMBED_DIM, E_PAD), lambda i: (0, 0)),
        ],
        out_specs=[
            pl.BlockSpec((BSZ, S_BLOCK, TOP_K), lambda i: (0, i, 0)),
            pl.BlockSpec((BSZ, S_BLOCK, TOP_K), lambda i: (0, i, 0)),
            pl.BlockSpec((1, 1), lambda i: (0, 0)),
        ],
        out_shape=[
            jax.ShapeDtypeStruct((BSZ, SEQ_LEN, TOP_K), jnp.int32),
            jax.ShapeDtypeStruct((BSZ, SEQ_LEN, TOP_K), jnp.float32),
            jax.ShapeDtypeStruct((1, 1), jnp.float32),
        ],
        scratch_shapes=[
            pltpu.VMEM((2, RB, EMBED_DIM), jnp.float32),
            pltpu.SemaphoreType.DMA((2, BSZ)),
            pltpu.VMEM((2, E_PAD), jnp.float32),
            pltpu.VMEM((2, E_PAD), jnp.float32),
        ],
    )(x, w_pad)
    idx = jnp.swapaxes(idx3, 0, 1).reshape(SEQ_LEN * BSZ, TOP_K)
    wgt = jnp.swapaxes(wgt3, 0, 1).reshape(SEQ_LEN * BSZ, TOP_K)
    return idx, wgt, aux[0, 0]


def kernel(x, weight):
    w_pad = jnp.zeros((EMBED_DIM, E_PAD), dtype=weight.dtype).at[:, :N_EXPERTS].set(weight.T)
    return _gate(x, w_pad)


# speedup vs baseline: 1.6410x; 1.0008x over previous
import functools

import jax
import jax.numpy as jnp
from jax.experimental import pallas as pl
from jax.experimental.pallas import tpu as pltpu

SEQ_LEN = 4096
BSZ = 2
EMBED_DIM = 4096
N_EXPERTS = 31
TOP_K = 3
ALPHA = 0.001

E_PAD = 128
S_BLOCK = 256
RB = S_BLOCK * BSZ
N_BLOCKS = SEQ_LEN // S_BLOCK
HALF_BLOCKS = (SEQ_LEN // 2) // S_BLOCK
NEG = -1e30


N_CH = 4
CH = S_BLOCK // N_CH


def _copy(x_hbm, x_buf, sem, step, slot, b, c):
    return pltpu.make_async_copy(
        x_hbm.at[pl.ds(step * S_BLOCK + c * CH, CH), b, :],
        x_buf.at[slot, pl.ds(b * S_BLOCK + c * CH, CH), :],
        sem.at[slot, b, c],
    )


def _start_copies(x_hbm, x_buf, sem, step, slot):
    for b in range(BSZ):
        for c in range(N_CH):
            _copy(x_hbm, x_buf, sem, step, slot, b, c).start()


def _wait_copies(x_hbm, x_buf, sem, step, slot):
    for b in range(BSZ):
        for c in range(N_CH):
            _copy(x_hbm, x_buf, sem, step, slot, b, c).wait()


def _gate_kernel(x_hbm, w_ref, idx_ref, wgt_ref, aux_ref, x_buf, sem,
                 cnt_acc, sum_acc):
    i = pl.program_id(0)
    slot = jax.lax.rem(i, 2)

    @pl.when(i == 0)
    def _init():
        cnt_acc[...] = jnp.zeros_like(cnt_acc)
        sum_acc[...] = jnp.zeros_like(sum_acc)
        _start_copies(x_hbm, x_buf, sem, 0, 0)

    @pl.when(i + 1 < N_BLOCKS)
    def _prefetch():
        _start_copies(x_hbm, x_buf, sem, i + 1, 1 - slot)

    _wait_copies(x_hbm, x_buf, sem, i, slot)

    xb = x_buf[slot]
    logits = jax.lax.dot_general(
        xb, w_ref[...],
        dimension_numbers=(((1,), (0,)), ((), ())),
        preferred_element_type=jnp.float32,
        precision=jax.lax.Precision.DEFAULT,
    )
    lane = jax.lax.broadcasted_iota(jnp.int32, (RB, E_PAD), 1)
    logits = jnp.where(lane < N_EXPERTS, logits, NEG)

    m = jnp.max(logits, axis=-1, keepdims=True)
    p = jnp.exp(logits - m)
    scores = p / jnp.sum(p, axis=-1, keepdims=True)

    cur = scores
    vals = []
    idxs = []
    for _ in range(TOP_K):
        v = jnp.max(cur, axis=-1, keepdims=True)
        hit = cur >= v
        ix = jnp.min(jnp.where(hit, lane, E_PAD), axis=-1, keepdims=True)
        vals.append(v)
        idxs.append(ix)
        cur = jnp.where(lane == ix, -1.0, cur)

    topv = jnp.concatenate(vals, axis=-1)
    topi = jnp.concatenate(idxs, axis=-1)
    denom = jnp.sum(topv, axis=-1, keepdims=True) + 1e-20
    idx_ref[...] = topi.reshape(BSZ, S_BLOCK, TOP_K)
    wgt_ref[...] = (topv / denom).reshape(BSZ, S_BLOCK, TOP_K)

    h = i // HALF_BLOCKS
    onehot_h = (jax.lax.broadcasted_iota(jnp.int32, (2, 1), 0) == h).astype(jnp.float32)

    sum_acc[...] += onehot_h * jnp.sum(scores, axis=0)[None, :]
    cnt = jnp.zeros((E_PAD,), dtype=jnp.float32)
    for k in range(TOP_K):
        cnt += jnp.sum((lane == topi[:, k:k + 1]).astype(jnp.float32), axis=0)
    cnt_acc[...] += onehot_h * cnt[None, :]

    @pl.when(i == N_BLOCKS - 1)
    def _finalize():
        scale = ALPHA * (1.0 / BSZ) * N_EXPERTS / (SEQ_LEN * SEQ_LEN * TOP_K)
        aux_ref[...] = (jnp.sum(cnt_acc[...] * sum_acc[...]) * scale).reshape(1, 1)


@functools.partial(jax.jit, static_argnums=())
def _gate(x, w_pad):
    idx3, wgt3, aux = pl.pallas_call(
        _gate_kernel,
        grid=(N_BLOCKS,),
        in_specs=[
            pl.BlockSpec(memory_space=pl.ANY),
            pl.BlockSpec((EMBED_DIM, E_PAD), lambda i: (0, 0)),
        ],
        out_specs=[
            pl.BlockSpec((BSZ, S_BLOCK, TOP_K), lambda i: (0, i, 0)),
            pl.BlockSpec((BSZ, S_BLOCK, TOP_K), lambda i: (0, i, 0)),
            pl.BlockSpec((1, 1), lambda i: (0, 0)),
        ],
        out_shape=[
            jax.ShapeDtypeStruct((BSZ, SEQ_LEN, TOP_K), jnp.int32),
            jax.ShapeDtypeStruct((BSZ, SEQ_LEN, TOP_K), jnp.float32),
            jax.ShapeDtypeStruct((1, 1), jnp.float32),
        ],
        scratch_shapes=[
            pltpu.VMEM((2, RB, EMBED_DIM), jnp.float32),
            pltpu.SemaphoreType.DMA((2, BSZ, N_CH)),
            pltpu.VMEM((2, E_PAD), jnp.float32),
            pltpu.VMEM((2, E_PAD), jnp.float32),
        ],
    )(x, w_pad)
    idx = jnp.swapaxes(idx3, 0, 1).reshape(SEQ_LEN * BSZ, TOP_K)
    wgt = jnp.swapaxes(wgt3, 0, 1).reshape(SEQ_LEN * BSZ, TOP_K)
    return idx, wgt, aux[0, 0]


def kernel(x, weight):
    w_pad = jnp.zeros((EMBED_DIM, E_PAD), dtype=weight.dtype).at[:, :N_EXPERTS].set(weight.T)
    return _gate(x, w_pad)


# speedup vs baseline: 3.1925x; 1.9455x over previous
import functools

import jax
import jax.numpy as jnp
from jax.experimental import pallas as pl
from jax.experimental.pallas import tpu as pltpu

SEQ_LEN = 4096
BSZ = 2
EMBED_DIM = 4096
N_EXPERTS = 31
TOP_K = 3
ALPHA = 0.001

E_PAD = 128
S_BLOCK = 256
RB = S_BLOCK * BSZ
N_BLOCKS = SEQ_LEN // S_BLOCK
HALF_BLOCKS = (SEQ_LEN // 2) // S_BLOCK
NEG = -1e30


def _copy(x_hbm, x_buf, sem, step, slot):
    x2d = x_hbm.reshape(SEQ_LEN * BSZ, EMBED_DIM)
    return pltpu.make_async_copy(
        x2d.at[pl.ds(step * RB, RB), :],
        x_buf.at[slot],
        sem.at[slot],
    )


def _start_copies(x_hbm, x_buf, sem, step, slot):
    _copy(x_hbm, x_buf, sem, step, slot).start()


def _wait_copies(x_hbm, x_buf, sem, step, slot):
    _copy(x_hbm, x_buf, sem, step, slot).wait()


def _gate_kernel(x_hbm, w_ref, idx_ref, wgt_ref, aux_ref, x_buf, sem,
                 cnt_acc, sum_acc):
    i = pl.program_id(0)
    slot = jax.lax.rem(i, 2)

    @pl.when(i == 0)
    def _init():
        cnt_acc[...] = jnp.zeros_like(cnt_acc)
        sum_acc[...] = jnp.zeros_like(sum_acc)
        _start_copies(x_hbm, x_buf, sem, 0, 0)

    @pl.when(i + 1 < N_BLOCKS)
    def _prefetch():
        _start_copies(x_hbm, x_buf, sem, i + 1, 1 - slot)

    _wait_copies(x_hbm, x_buf, sem, i, slot)

    xb = x_buf[slot]
    logits = jax.lax.dot_general(
        xb, w_ref[...],
        dimension_numbers=(((1,), (0,)), ((), ())),
        preferred_element_type=jnp.float32,
        precision=jax.lax.Precision.DEFAULT,
    )
    lane = jax.lax.broadcasted_iota(jnp.int32, (RB, E_PAD), 1)
    logits = jnp.where(lane < N_EXPERTS, logits, NEG)

    m = jnp.max(logits, axis=-1, keepdims=True)
    p = jnp.exp(logits - m)
    scores = p / jnp.sum(p, axis=-1, keepdims=True)

    cur = scores
    vals = []
    idxs = []
    for _ in range(TOP_K):
        v = jnp.max(cur, axis=-1, keepdims=True)
        hit = cur >= v
        ix = jnp.min(jnp.where(hit, lane, E_PAD), axis=-1, keepdims=True)
        vals.append(v)
        idxs.append(ix)
        cur = jnp.where(lane == ix, -1.0, cur)

    topv = jnp.concatenate(vals, axis=-1)
    topi = jnp.concatenate(idxs, axis=-1)
    denom = jnp.sum(topv, axis=-1, keepdims=True) + 1e-20
    idx_ref[...] = topi
    wgt_ref[...] = topv / denom

    h = i // HALF_BLOCKS
    onehot_h = (jax.lax.broadcasted_iota(jnp.int32, (2, 1), 0) == h).astype(jnp.float32)

    sum_acc[...] += onehot_h * jnp.sum(scores, axis=0)[None, :]
    cnt = jnp.zeros((E_PAD,), dtype=jnp.float32)
    for k in range(TOP_K):
        cnt += jnp.sum((lane == topi[:, k:k + 1]).astype(jnp.float32), axis=0)
    cnt_acc[...] += onehot_h * cnt[None, :]

    @pl.when(i == N_BLOCKS - 1)
    def _finalize():
        scale = ALPHA * (1.0 / BSZ) * N_EXPERTS / (SEQ_LEN * SEQ_LEN * TOP_K)
        aux_ref[...] = (jnp.sum(cnt_acc[...] * sum_acc[...]) * scale).reshape(1, 1)


@functools.partial(jax.jit, static_argnums=())
def _gate(x, w_pad):
    idx3, wgt3, aux = pl.pallas_call(
        _gate_kernel,
        grid=(N_BLOCKS,),
        in_specs=[
            pl.BlockSpec(memory_space=pl.ANY),
            pl.BlockSpec((EMBED_DIM, E_PAD), lambda i: (0, 0)),
        ],
        out_specs=[
            pl.BlockSpec((RB, TOP_K), lambda i: (i, 0)),
            pl.BlockSpec((RB, TOP_K), lambda i: (i, 0)),
            pl.BlockSpec((1, 1), lambda i: (0, 0)),
        ],
        out_shape=[
            jax.ShapeDtypeStruct((SEQ_LEN * BSZ, TOP_K), jnp.int32),
            jax.ShapeDtypeStruct((SEQ_LEN * BSZ, TOP_K), jnp.float32),
            jax.ShapeDtypeStruct((1, 1), jnp.float32),
        ],
        scratch_shapes=[
            pltpu.VMEM((2, RB, EMBED_DIM), jnp.float32),
            pltpu.SemaphoreType.DMA((2,)),
            pltpu.VMEM((2, E_PAD), jnp.float32),
            pltpu.VMEM((2, E_PAD), jnp.float32),
        ],
    )(x, w_pad)
    return idx3, wgt3, aux[0, 0]


def kernel(x, weight):
    w_pad = jnp.zeros((EMBED_DIM, E_PAD), dtype=weight.dtype).at[:, :N_EXPERTS].set(weight.T)
    return _gate(x, w_pad)


# speedup vs baseline: 4.3607x; 1.3659x over previous
import functools

import jax
import jax.numpy as jnp
from jax.experimental import pallas as pl
from jax.experimental.pallas import tpu as pltpu

SEQ_LEN = 4096
BSZ = 2
EMBED_DIM = 4096
N_EXPERTS = 31
TOP_K = 3
ALPHA = 0.001

E_PAD = 128
RB = 512
N_ROWS = SEQ_LEN * BSZ
N_BLOCKS = N_ROWS // RB
HALF_BLOCKS = (N_ROWS // 2) // RB
NEG = -1e30


def _copy(x_hbm, x_buf, sem, step, slot):
    x2d = x_hbm.reshape(N_ROWS, EMBED_DIM)
    return pltpu.make_async_copy(
        x2d.at[pl.ds(step * RB, RB), :],
        x_buf.at[slot],
        sem.at[slot],
    )


def _gate_kernel(x_hbm, w_ref, idx_ref, wgt_ref, aux_ref, x_buf, sem,
                 cnt_acc, sum_acc):
    i = pl.program_id(0)
    slot = jax.lax.rem(i, 2)

    @pl.when(i == 0)
    def _init():
        cnt_acc[...] = jnp.zeros_like(cnt_acc)
        sum_acc[...] = jnp.zeros_like(sum_acc)
        _copy(x_hbm, x_buf, sem, 0, 0).start()

    @pl.when(i + 1 < N_BLOCKS)
    def _prefetch():
        _copy(x_hbm, x_buf, sem, i + 1, 1 - slot).start()

    _copy(x_hbm, x_buf, sem, i, slot).wait()

    logits = jax.lax.dot_general(
        w_ref[...], x_buf[slot],
        dimension_numbers=(((0,), (1,)), ((), ())),
        preferred_element_type=jnp.float32,
        precision=jax.lax.Precision.DEFAULT,
    )
    sub = jax.lax.broadcasted_iota(jnp.int32, (E_PAD, RB), 0)
    logits = jnp.where(sub < N_EXPERTS, logits, NEG)

    m = jnp.max(logits, axis=0, keepdims=True)
    p = jnp.exp(logits - m)
    z = jnp.sum(p, axis=0, keepdims=True)

    cur = logits
    vals = []
    idxs = []
    for _ in range(TOP_K):
        v = jnp.max(cur, axis=0, keepdims=True)
        hit = cur >= v
        ix = jnp.min(jnp.where(hit, sub, E_PAD), axis=0, keepdims=True)
        vals.append(v)
        idxs.append(ix)
        cur = jnp.where(sub == ix, NEG, cur)

    ts = [jnp.exp(v - m) / z for v in vals]
    denom = ts[0] + ts[1] + ts[2] + 1e-20
    idx_ref[...] = jnp.concatenate(idxs, axis=0)
    wgt_ref[...] = jnp.concatenate([t / denom for t in ts], axis=0)

    h = i // HALF_BLOCKS
    onehot_h = (jax.lax.broadcasted_iota(jnp.int32, (1, 2), 1) == h).astype(jnp.float32)

    scores_sum = jnp.sum(p / z, axis=1, keepdims=True)
    sum_acc[...] += scores_sum * onehot_h
    cnt = jnp.zeros((E_PAD, 1), dtype=jnp.float32)
    for k in range(TOP_K):
        cnt += jnp.sum((sub == idxs[k]).astype(jnp.float32), axis=1, keepdims=True)
    cnt_acc[...] += cnt * onehot_h

    @pl.when(i == N_BLOCKS - 1)
    def _finalize():
        scale = ALPHA * (1.0 / BSZ) * N_EXPERTS / (SEQ_LEN * SEQ_LEN * TOP_K)
        aux_ref[...] = (jnp.sum(cnt_acc[...] * sum_acc[...]) * scale).reshape(1, 1)


@functools.partial(jax.jit, static_argnums=())
def _gate(x, w_pad):
    idxT, wgtT, aux = pl.pallas_call(
        _gate_kernel,
        grid=(N_BLOCKS,),
        in_specs=[
            pl.BlockSpec(memory_space=pl.ANY),
            pl.BlockSpec((EMBED_DIM, E_PAD), lambda i: (0, 0)),
        ],
        out_specs=[
            pl.BlockSpec((TOP_K, RB), lambda i: (0, i)),
            pl.BlockSpec((TOP_K, RB), lambda i: (0, i)),
            pl.BlockSpec((1, 1), lambda i: (0, 0)),
        ],
        out_shape=[
            jax.ShapeDtypeStruct((TOP_K, N_ROWS), jnp.int32),
            jax.ShapeDtypeStruct((TOP_K, N_ROWS), jnp.float32),
            jax.ShapeDtypeStruct((1, 1), jnp.float32),
        ],
        scratch_shapes=[
            pltpu.VMEM((2, RB, EMBED_DIM), jnp.float32),
            pltpu.SemaphoreType.DMA((2,)),
            pltpu.VMEM((E_PAD, 2), jnp.float32),
            pltpu.VMEM((E_PAD, 2), jnp.float32),
        ],
    )(x, w_pad)
    return idxT.T, wgtT.T, aux[0, 0]


def kernel(x, weight):
    w_pad = jnp.zeros((EMBED_DIM, E_PAD), dtype=weight.dtype).at[:, :N_EXPERTS].set(weight.T)
    return _gate(x, w_pad)


# speedup vs baseline: 4.7000x; 1.0778x over previous
import functools

import jax
import jax.numpy as jnp
from jax.experimental import pallas as pl
from jax.experimental.pallas import tpu as pltpu

SEQ_LEN = 4096
BSZ = 2
EMBED_DIM = 4096
N_EXPERTS = 31
TOP_K = 3
ALPHA = 0.001

E_PAD = 32
RB = 512
N_ROWS = SEQ_LEN * BSZ
N_BLOCKS = N_ROWS // RB
HALF_BLOCKS = (N_ROWS // 2) // RB
NEG = -1e30


def _copy(x_hbm, x_buf, sem, step, slot):
    x2d = x_hbm.reshape(N_ROWS, EMBED_DIM)
    return pltpu.make_async_copy(
        x2d.at[pl.ds(step * RB, RB), :],
        x_buf.at[slot],
        sem.at[slot],
    )


def _gate_kernel(x_hbm, w_ref, idx_ref, wgt_ref, aux_ref, x_buf, sem,
                 cnt_acc, sum_acc):
    i = pl.program_id(0)
    slot = jax.lax.rem(i, 2)

    @pl.when(i == 0)
    def _init():
        cnt_acc[...] = jnp.zeros_like(cnt_acc)
        sum_acc[...] = jnp.zeros_like(sum_acc)
        _copy(x_hbm, x_buf, sem, 0, 0).start()

    @pl.when(i + 1 < N_BLOCKS)
    def _prefetch():
        _copy(x_hbm, x_buf, sem, i + 1, 1 - slot).start()

    _copy(x_hbm, x_buf, sem, i, slot).wait()

    logits = jax.lax.dot_general(
        w_ref[...], x_buf[slot],
        dimension_numbers=(((0,), (1,)), ((), ())),
        preferred_element_type=jnp.float32,
        precision=jax.lax.Precision.DEFAULT,
    )
    sub = jax.lax.broadcasted_iota(jnp.int32, (E_PAD, RB), 0)
    logits = jnp.where(sub < N_EXPERTS, logits, NEG)

    m = jnp.max(logits, axis=0, keepdims=True)
    p = jnp.exp(logits - m)
    z = jnp.sum(p, axis=0, keepdims=True)

    cur = logits
    vals = []
    idxs = []
    for _ in range(TOP_K):
        v = jnp.max(cur, axis=0, keepdims=True)
        hit = cur >= v
        ix = jnp.min(jnp.where(hit, sub, E_PAD), axis=0, keepdims=True)
        vals.append(v)
        idxs.append(ix)
        cur = jnp.where(sub == ix, NEG, cur)

    ts = [jnp.exp(v - m) / z for v in vals]
    denom = ts[0] + ts[1] + ts[2] + 1e-20
    idx_ref[...] = jnp.concatenate(idxs, axis=0)
    wgt_ref[...] = jnp.concatenate([t / denom for t in ts], axis=0)

    h = i // HALF_BLOCKS
    onehot_h = (jax.lax.broadcasted_iota(jnp.int32, (1, 2), 1) == h).astype(jnp.float32)

    scores_sum = jnp.sum(p / z, axis=1, keepdims=True)
    sum_acc[...] += scores_sum * onehot_h
    cnt = jnp.zeros((E_PAD, 1), dtype=jnp.float32)
    for k in range(TOP_K):
        cnt += jnp.sum((sub == idxs[k]).astype(jnp.float32), axis=1, keepdims=True)
    cnt_acc[...] += cnt * onehot_h

    @pl.when(i == N_BLOCKS - 1)
    def _finalize():
        scale = ALPHA * (1.0 / BSZ) * N_EXPERTS / (SEQ_LEN * SEQ_LEN * TOP_K)
        aux_ref[...] = (jnp.sum(cnt_acc[...] * sum_acc[...]) * scale).reshape(1, 1)


@functools.partial(jax.jit, static_argnums=())
def _gate(x, w_pad):
    idxT, wgtT, aux = pl.pallas_call(
        _gate_kernel,
        grid=(N_BLOCKS,),
        in_specs=[
            pl.BlockSpec(memory_space=pl.ANY),
            pl.BlockSpec((EMBED_DIM, E_PAD), lambda i: (0, 0)),
        ],
        out_specs=[
            pl.BlockSpec((TOP_K, RB), lambda i: (0, i)),
            pl.BlockSpec((TOP_K, RB), lambda i: (0, i)),
            pl.BlockSpec((1, 1), lambda i: (0, 0)),
        ],
        out_shape=[
            jax.ShapeDtypeStruct((TOP_K, N_ROWS), jnp.int32),
            jax.ShapeDtypeStruct((TOP_K, N_ROWS), jnp.float32),
            jax.ShapeDtypeStruct((1, 1), jnp.float32),
        ],
        scratch_shapes=[
            pltpu.VMEM((2, RB, EMBED_DIM), jnp.float32),
            pltpu.SemaphoreType.DMA((2,)),
            pltpu.VMEM((E_PAD, 2), jnp.float32),
            pltpu.VMEM((E_PAD, 2), jnp.float32),
        ],
    )(x, w_pad)
    return idxT.T, wgtT.T, aux[0, 0]


def kernel(x, weight):
    w_pad = jnp.zeros((EMBED_DIM, E_PAD), dtype=weight.dtype).at[:, :N_EXPERTS].set(weight.T)
    return _gate(x, w_pad)


# speedup vs baseline: 4.8726x; 1.0367x over previous
import functools

import jax
import jax.numpy as jnp
from jax.experimental import pallas as pl
from jax.experimental.pallas import tpu as pltpu

SEQ_LEN = 4096
BSZ = 2
EMBED_DIM = 4096
N_EXPERTS = 31
TOP_K = 3
ALPHA = 0.001

E_PAD = 32
RB = 512
N_ROWS = SEQ_LEN * BSZ
N_BLOCKS = N_ROWS // RB
HALF_BLOCKS = (N_ROWS // 2) // RB
NEG = -1e30


def _copy(x_hbm, x_buf, sem, step, slot):
    x2d = x_hbm.reshape(N_ROWS, EMBED_DIM)
    return pltpu.make_async_copy(
        x2d.at[pl.ds(step * RB, RB), :],
        x_buf.at[slot],
        sem.at[slot],
    )


def _gate_kernel(x_hbm, w_ref, idx_ref, wgt_ref, aux_ref, x_buf, sem,
                 cnt_acc, sum_acc):
    i = pl.program_id(0)
    slot = jax.lax.rem(i, 2)

    @pl.when(i == 0)
    def _init():
        cnt_acc[...] = jnp.zeros_like(cnt_acc)
        sum_acc[...] = jnp.zeros_like(sum_acc)
        _copy(x_hbm, x_buf, sem, 0, 0).start()

    @pl.when(i + 1 < N_BLOCKS)
    def _prefetch():
        _copy(x_hbm, x_buf, sem, i + 1, 1 - slot).start()

    _copy(x_hbm, x_buf, sem, i, slot).wait()

    logits = jax.lax.dot_general(
        w_ref[...], x_buf[slot],
        dimension_numbers=(((1,), (1,)), ((), ())),
        preferred_element_type=jnp.float32,
        precision=jax.lax.Precision.DEFAULT,
    )
    sub = jax.lax.broadcasted_iota(jnp.int32, (E_PAD, RB), 0)
    logits = jnp.where(sub < N_EXPERTS, logits, NEG)

    m = jnp.max(logits, axis=0, keepdims=True)
    p = jnp.exp(logits - m)
    z = jnp.sum(p, axis=0, keepdims=True)

    cur = logits
    vals = []
    idxs = []
    for _ in range(TOP_K):
        v = jnp.max(cur, axis=0, keepdims=True)
        hit = cur >= v
        ix = jnp.min(jnp.where(hit, sub, E_PAD), axis=0, keepdims=True)
        vals.append(v)
        idxs.append(ix)
        cur = jnp.where(sub == ix, NEG, cur)

    ts = [jnp.exp(v - m) / z for v in vals]
    denom = ts[0] + ts[1] + ts[2] + 1e-20
    idx_ref[...] = jnp.concatenate(idxs, axis=0)
    wgt_ref[...] = jnp.concatenate([t / denom for t in ts], axis=0)

    h = i // HALF_BLOCKS
    onehot_h = (jax.lax.broadcasted_iota(jnp.int32, (1, 2), 1) == h).astype(jnp.float32)

    scores_sum = jnp.sum(p / z, axis=1, keepdims=True)
    sum_acc[...] += scores_sum * onehot_h
    cnt = jnp.zeros((E_PAD, 1), dtype=jnp.float32)
    for k in range(TOP_K):
        cnt += jnp.sum((sub == idxs[k]).astype(jnp.float32), axis=1, keepdims=True)
    cnt_acc[...] += cnt * onehot_h

    @pl.when(i == N_BLOCKS - 1)
    def _finalize():
        scale = ALPHA * (1.0 / BSZ) * N_EXPERTS / (SEQ_LEN * SEQ_LEN * TOP_K)
        aux_ref[...] = (jnp.sum(cnt_acc[...] * sum_acc[...]) * scale).reshape(1, 1)


@functools.partial(jax.jit, static_argnums=())
def _gate(x, w_pad):
    idxT, wgtT, aux = pl.pallas_call(
        _gate_kernel,
        grid=(N_BLOCKS,),
        in_specs=[
            pl.BlockSpec(memory_space=pl.ANY),
            pl.BlockSpec((E_PAD, EMBED_DIM), lambda i: (0, 0)),
        ],
        out_specs=[
            pl.BlockSpec((TOP_K, RB), lambda i: (0, i)),
            pl.BlockSpec((TOP_K, RB), lambda i: (0, i)),
            pl.BlockSpec((1, 1), lambda i: (0, 0)),
        ],
        out_shape=[
            jax.ShapeDtypeStruct((TOP_K, N_ROWS), jnp.int32),
            jax.ShapeDtypeStruct((TOP_K, N_ROWS), jnp.float32),
            jax.ShapeDtypeStruct((1, 1), jnp.float32),
        ],
        scratch_shapes=[
            pltpu.VMEM((2, RB, EMBED_DIM), jnp.float32),
            pltpu.SemaphoreType.DMA((2,)),
            pltpu.VMEM((E_PAD, 2), jnp.float32),
            pltpu.VMEM((E_PAD, 2), jnp.float32),
        ],
    )(x, w_pad)
    return idxT.T, wgtT.T, aux[0, 0]


def kernel(x, weight):
    w_pad = jnp.zeros((E_PAD, EMBED_DIM), dtype=weight.dtype).at[:N_EXPERTS].set(weight)
    return _gate(x, w_pad)


# speedup vs baseline: 4.8732x; 1.0001x over previous
import functools

import jax
import jax.numpy as jnp
from jax.experimental import pallas as pl
from jax.experimental.pallas import tpu as pltpu

SEQ_LEN = 4096
BSZ = 2
EMBED_DIM = 4096
N_EXPERTS = 31
TOP_K = 3
ALPHA = 0.001

E_PAD = 32
RB = 1024
N_ROWS = SEQ_LEN * BSZ
N_BLOCKS = N_ROWS // RB
HALF_BLOCKS = (N_ROWS // 2) // RB
NEG = -1e30


def _copy(x_hbm, x_buf, sem, step, slot):
    x2d = x_hbm.reshape(N_ROWS, EMBED_DIM)
    return pltpu.make_async_copy(
        x2d.at[pl.ds(step * RB, RB), :],
        x_buf.at[slot],
        sem.at[slot],
    )


def _gate_kernel(x_hbm, w_ref, idx_ref, wgt_ref, aux_ref, x_buf, sem,
                 cnt_acc, sum_acc):
    i = pl.program_id(0)
    slot = jax.lax.rem(i, 2)

    @pl.when(i == 0)
    def _init():
        cnt_acc[...] = jnp.zeros_like(cnt_acc)
        sum_acc[...] = jnp.zeros_like(sum_acc)
        _copy(x_hbm, x_buf, sem, 0, 0).start()

    @pl.when(i + 1 < N_BLOCKS)
    def _prefetch():
        _copy(x_hbm, x_buf, sem, i + 1, 1 - slot).start()

    _copy(x_hbm, x_buf, sem, i, slot).wait()

    logits = jax.lax.dot_general(
        w_ref[...], x_buf[slot],
        dimension_numbers=(((1,), (1,)), ((), ())),
        preferred_element_type=jnp.float32,
        precision=jax.lax.Precision.DEFAULT,
    )
    sub = jax.lax.broadcasted_iota(jnp.int32, (E_PAD, RB), 0)
    logits = jnp.where(sub < N_EXPERTS, logits, NEG)

    m = jnp.max(logits, axis=0, keepdims=True)
    p = jnp.exp(logits - m)
    z = jnp.sum(p, axis=0, keepdims=True)

    cur = logits
    vals = []
    idxs = []
    for _ in range(TOP_K):
        v = jnp.max(cur, axis=0, keepdims=True)
        hit = cur >= v
        ix = jnp.min(jnp.where(hit, sub, E_PAD), axis=0, keepdims=True)
        vals.append(v)
        idxs.append(ix)
        cur = jnp.where(sub == ix, NEG, cur)

    ts = [jnp.exp(v - m) / z for v in vals]
    denom = ts[0] + ts[1] + ts[2] + 1e-20
    idx_ref[...] = jnp.concatenate(idxs, axis=0)
    wgt_ref[...] = jnp.concatenate([t / denom for t in ts], axis=0)

    h = i // HALF_BLOCKS
    onehot_h = (jax.lax.broadcasted_iota(jnp.int32, (1, 2), 1) == h).astype(jnp.float32)

    scores_sum = jnp.sum(p / z, axis=1, keepdims=True)
    sum_acc[...] += scores_sum * onehot_h
    cnt = jnp.zeros((E_PAD, 1), dtype=jnp.float32)
    for k in range(TOP_K):
        cnt += jnp.sum((sub == idxs[k]).astype(jnp.float32), axis=1, keepdims=True)
    cnt_acc[...] += cnt * onehot_h

    @pl.when(i == N_BLOCKS - 1)
    def _finalize():
        scale = ALPHA * (1.0 / BSZ) * N_EXPERTS / (SEQ_LEN * SEQ_LEN * TOP_K)
        aux_ref[...] = (jnp.sum(cnt_acc[...] * sum_acc[...]) * scale).reshape(1, 1)


@functools.partial(jax.jit, static_argnums=())
def _gate(x, w_pad):
    idxT, wgtT, aux = pl.pallas_call(
        _gate_kernel,
        grid=(N_BLOCKS,),
        in_specs=[
            pl.BlockSpec(memory_space=pl.ANY),
            pl.BlockSpec((E_PAD, EMBED_DIM), lambda i: (0, 0)),
        ],
        out_specs=[
            pl.BlockSpec((TOP_K, RB), lambda i: (0, i)),
            pl.BlockSpec((TOP_K, RB), lambda i: (0, i)),
            pl.BlockSpec((1, 1), lambda i: (0, 0)),
        ],
        out_shape=[
            jax.ShapeDtypeStruct((TOP_K, N_ROWS), jnp.int32),
            jax.ShapeDtypeStruct((TOP_K, N_ROWS), jnp.float32),
            jax.ShapeDtypeStruct((1, 1), jnp.float32),
        ],
        scratch_shapes=[
            pltpu.VMEM((2, RB, EMBED_DIM), jnp.float32),
            pltpu.SemaphoreType.DMA((2,)),
            pltpu.VMEM((E_PAD, 2), jnp.float32),
            pltpu.VMEM((E_PAD, 2), jnp.float32),
        ],
    )(x, w_pad)
    return idxT.T, wgtT.T, aux[0, 0]


def kernel(x, weight):
    w_pad = jnp.zeros((E_PAD, EMBED_DIM), dtype=weight.dtype).at[:N_EXPERTS].set(weight)
    return _gate(x, w_pad)
